# Initial kernel scaffold; baseline (speedup 1.0000x reference)
import jax
import jax.numpy as jnp
from jax.experimental import pallas as pl


def kernel(word_sequences, table):
    raise NotImplementedError("write your pallas kernel here")


# speedup vs baseline: 3.3287x; 3.3287x over previous
import functools

import jax
import jax.numpy as jnp
from jax import lax
from jax.experimental import pallas as pl
from jax.experimental.pallas import tpu as pltpu
from jax.experimental.pallas import tpu_sc as plsc

EMB = 128
NC = 2
NS = 16
NW = NC * NS
CHUNK = 128


def _make_gather(n_rows: int):
    assert n_rows % (NW * CHUNK) == 0
    rows_per_w = n_rows // NW
    n_chunks = rows_per_w // CHUNK
    mesh = plsc.VectorSubcoreMesh(core_axis_name="c", subcore_axis_name="s")

    @functools.partial(
        pl.kernel,
        mesh=mesh,
        out_type=jax.ShapeDtypeStruct((n_rows, EMB), jnp.float32),
        scratch_types=[
            pltpu.VMEM((n_chunks, CHUNK), jnp.int32),
            pltpu.VMEM((CHUNK, EMB), jnp.float32),
            pltpu.VMEM((CHUNK, EMB), jnp.float32),
            pltpu.SemaphoreType.DMA,
            pltpu.SemaphoreType.DMA,
        ],
    )
    def gather(table_hbm, idx_hbm, out_hbm, idx_v, rows_a, rows_b, sem_a, sem_b):
        wid = lax.axis_index("s") * NC + lax.axis_index("c")
        base = wid * rows_per_w
        pltpu.sync_copy(idx_hbm.at[wid], idx_v)

        bufs = (rows_a, rows_b)
        sems = (sem_a, sem_b)

        pltpu.async_copy(table_hbm.at[idx_v.at[0]], rows_a, sem_a)

        def body(i, _):
            for b in range(2):
                c = i * 2 + b
                cur, nxt = bufs[b], bufs[1 - b]
                csem, nsem = sems[b], sems[1 - b]

                @pl.when(c + 1 < n_chunks)
                def _():
                    pltpu.async_copy(table_hbm.at[idx_v.at[c + 1]], nxt, nsem)

                pltpu.make_async_copy(
                    table_hbm.at[idx_v.at[c]], cur, csem
                ).wait()
                pltpu.sync_copy(cur, out_hbm.at[pl.ds(base + c * CHUNK, CHUNK)])
            return 0

        lax.fori_loop(0, n_chunks // 2, body, 0)

    return gather


@jax.jit
def kernel(word_sequences, table):
    n_seq, seq_len = word_sequences.shape
    n_rows = n_seq * seq_len
    idx = word_sequences.astype(jnp.int32).reshape(NW, n_rows // (NW * CHUNK), CHUNK)
    out = _make_gather(n_rows)(table, idx)
    return out.reshape(n_seq, seq_len, EMB)


# speedup vs baseline: 3.3422x; 1.0041x over previous
import functools

import jax
import jax.numpy as jnp
from jax import lax
from jax.experimental import pallas as pl
from jax.experimental.pallas import tpu as pltpu
from jax.experimental.pallas import tpu_sc as plsc

EMB = 128
NC = 2
NS = 16
NW = NC * NS
CHUNK = 128


DEPTH = 4


def _make_gather(n_rows: int):
    assert n_rows % (NW * CHUNK) == 0
    rows_per_w = n_rows // NW
    n_chunks = rows_per_w // CHUNK
    assert n_chunks % DEPTH == 2 or n_chunks % DEPTH == 0
    tail = n_chunks % DEPTH
    mesh = plsc.VectorSubcoreMesh(core_axis_name="c", subcore_axis_name="s")

    @functools.partial(
        pl.kernel,
        mesh=mesh,
        out_type=jax.ShapeDtypeStruct((n_rows, EMB), jnp.float32),
        scratch_types=[
            pltpu.VMEM((n_chunks, CHUNK), jnp.int32),
        ]
        + [pltpu.VMEM((CHUNK, EMB), jnp.float32) for _ in range(DEPTH)]
        + [pltpu.SemaphoreType.DMA for _ in range(2 * DEPTH)],
    )
    def gather(table_hbm, idx_hbm, out_hbm, idx_v, *rest):
        bufs = rest[:DEPTH]
        gsems = rest[DEPTH : 2 * DEPTH]
        osems = rest[2 * DEPTH :]
        wid = lax.axis_index("s") * NC + lax.axis_index("c")
        base = wid * rows_per_w
        pltpu.sync_copy(idx_hbm.at[wid], idx_v)

        def start_gather(c, b):
            pltpu.async_copy(table_hbm.at[idx_v.at[c]], bufs[b], gsems[b])

        def wait_gather(c, b):
            pltpu.make_async_copy(table_hbm.at[idx_v.at[c]], bufs[b], gsems[b]).wait()

        def out_slice(c):
            return out_hbm.at[pl.ds(base + c * CHUNK, CHUNK)]

        def start_out(c, b):
            pltpu.async_copy(bufs[b], out_slice(c), osems[b])

        def wait_out(c, b):
            pltpu.make_async_copy(bufs[b], out_slice(c), osems[b]).wait()

        for b in range(DEPTH):
            start_gather(b, b)

        def body(i, _):
            for b in range(DEPTH):
                c = i * DEPTH + b
                wait_gather(c, b)
                start_out(c, b)

                @pl.when(c + DEPTH < n_chunks)
                def _():
                    wait_out(c, b)
                    start_gather(c + DEPTH, b)

            return 0

        lax.fori_loop(0, n_chunks // DEPTH, body, 0)

        for t in range(tail):
            c = (n_chunks // DEPTH) * DEPTH + t
            wait_gather(c, t)
            start_out(c, t)

        for t in range(DEPTH):
            c = n_chunks - DEPTH + t
            wait_out(c, c % DEPTH)

    return gather


@jax.jit
def kernel(word_sequences, table):
    n_seq, seq_len = word_sequences.shape
    n_rows = n_seq * seq_len
    idx = word_sequences.astype(jnp.int32).reshape(NW, n_rows // (NW * CHUNK), CHUNK)
    out = _make_gather(n_rows)(table, idx)
    return out.reshape(n_seq, seq_len, EMB)
